# baseline (speedup 1.0000x reference)
import functools

import jax
import jax.numpy as jnp
import numpy as np
from jax import lax
from jax.experimental import pallas as pl
from jax.experimental.pallas import tpu as pltpu
from jax.experimental.pallas import tpu_sc as plsc

_N_FEAT = 2000
_BATCH = 8192

_NC = 2
_NS = 16
_NW = _NC * _NS
_N_TR = _N_FEAT // 8
_MAXK = 8

_SWAP_DST_SRC_2 = [
    (15, 462), (25, 1930), (26, 147), (43, 119), (59, 1744), (61, 964), (90, 167), (95, 1721),
    (114, 1123), (119, 43), (122, 1138), (126, 1965), (131, 426), (141, 908), (145, 1215), (147, 26),
    (156, 1437), (157, 1721), (167, 90), (173, 1276), (183, 997), (193, 1044), (227, 1925), (237, 1982),
    (242, 1400), (254, 1994), (273, 1139), (274, 1311), (277, 326), (286, 443), (318, 1618), (319, 1821),
    (326, 277), (338, 1058), (343, 996), (349, 1909), (357, 1013), (372, 1653), (380, 771), (393, 1834),
    (396, 683), (417, 669), (426, 131), (443, 286), (462, 15), (477, 616), (506, 969), (514, 1565),
    (515, 1103), (520, 1611), (550, 1224), (562, 1971), (566, 1333), (578, 1751), (582, 590), (584, 732),
    (589, 1894), (590, 582), (616, 477), (625, 1833), (634, 1623), (636, 646), (646, 636), (656, 1584),
    (669, 417), (680, 1724), (682, 1890), (683, 396), (690, 1617), (693, 1668), (699, 119), (732, 584),
    (741, 1807), (753, 1822), (754, 1162), (771, 380), (842, 879), (858, 1671), (879, 842), (888, 1800),
    (900, 1707), (908, 141), (911, 1515), (917, 1634), (927, 1948), (944, 1386), (945, 1022), (957, 1093),
    (961, 1330), (964, 61), (965, 1291), (969, 506), (996, 343), (997, 183), (1013, 357), (1022, 945),
    (1044, 193), (1058, 338), (1089, 1510), (1093, 957), (1103, 515), (1123, 114), (1138, 122), (1139, 273),
    (1162, 754), (1167, 1728), (1178, 1430), (1207, 1945), (1215, 145), (1224, 550), (1240, 1819), (1265, 1700),
    (1276, 173), (1291, 965), (1299, 1560), (1307, 1869), (1311, 274), (1323, 1968), (1330, 961), (1333, 566),
    (1348, 1497), (1359, 1607), (1381, 1412), (1386, 944), (1389, 1637), (1391, 1895), (1400, 242), (1412, 1381),
    (1417, 732), (1430, 1178), (1437, 156), (1455, 1644), (1465, 1651), (1480, 1752), (1497, 1348), (1510, 1089),
    (1515, 911), (1526, 1659), (1539, 1633), (1552, 1711), (1560, 1299), (1565, 514), (1584, 656), (1592, 1793),
    (1607, 1359), (1611, 520), (1617, 690), (1618, 318), (1623, 634), (1633, 1539), (1634, 917), (1637, 1389),
    (1644, 1455), (1651, 1465), (1653, 372), (1659, 1526), (1668, 693), (1671, 858), (1700, 1265), (1707, 900),
    (1711, 1552), (1721, 95), (1724, 680), (1728, 1167), (1744, 59), (1746, 1724), (1751, 578), (1752, 1480),
    (1793, 1592), (1800, 888), (1807, 741), (1819, 1240), (1821, 319), (1822, 753), (1833, 625), (1834, 393),
    (1869, 1307), (1890, 682), (1894, 589), (1895, 1391), (1909, 349), (1925, 227), (1930, 25), (1936, 1307),
    (1945, 1207), (1947, 147), (1948, 927), (1954, 242), (1965, 126), (1968, 1323), (1971, 562), (1982, 237),
    (1994, 254),
]


_W = 2048
_NCHUNK = _BATCH // _W
_NBUF = 3


def _idx_table(pairs):
    src_of = dict(pairs)
    ntrs = [_N_TR // _NW + (1 if w < _N_TR % _NW else 0) for w in range(_NW)]
    tab = np.zeros((_NW * 8, 8), dtype=np.int32)
    tr = 0
    for w in range(_NW):
        for k in range(ntrs[w]):
            t = tr + k
            tab[w * 8 + k] = [src_of.get(t * 8 + i, t * 8 + i)
                              for i in range(8)]
        tr += ntrs[w]
    return tab


_IDX_TAB = _idx_table(_SWAP_DST_SRC_2)
_NTR_BASE = _N_TR // _NW
_NTR_EXTRA = _N_TR % _NW

_mesh = plsc.VectorSubcoreMesh(core_axis_name="c", subcore_axis_name="s")


@functools.partial(
    pl.kernel,
    mesh=_mesh,
    compiler_params=pltpu.CompilerParams(
        needs_layout_passes=False, use_tc_tiling_on_sc=True),
    out_type=jax.ShapeDtypeStruct((_N_FEAT, _BATCH), jnp.float32),
    scratch_types=[
        pltpu.VMEM((8, _W), jnp.float32),
        pltpu.VMEM((8, _W), jnp.float32),
        pltpu.VMEM((8, _W), jnp.float32),
        pltpu.VMEM((8,), jnp.int32),
        pltpu.SemaphoreType.DMA,
        pltpu.SemaphoreType.DMA,
        pltpu.SemaphoreType.DMA,
        pltpu.SemaphoreType.DMA,
        pltpu.SemaphoreType.DMA,
        pltpu.SemaphoreType.DMA,
    ],
)
def _row_permute(x_hbm, idx_hbm, out_hbm,
                 buf_0, buf_1, buf_2, idx8_v,
                 isem_0, isem_1, isem_2, osem_0, osem_1, osem_2):
    cid = lax.axis_index("c")
    sid = lax.axis_index("s")
    wid = sid * _NC + cid
    ntr = _NTR_BASE + jnp.where(wid < _NTR_EXTRA, 1, 0)
    tr0 = _NTR_BASE * wid + jnp.minimum(wid, _NTR_EXTRA)

    bufs = (buf_0, buf_1, buf_2)
    isems = (isem_0, isem_1, isem_2)
    osems = (osem_0, osem_1, osem_2)

    def drain(b):
        pltpu.make_async_copy(x_hbm.at[pl.ds(0, 8), pl.ds(0, _W)], bufs[b],
                              osems[b]).wait()

    for k in range(_NTR_BASE + 1):
        is_last = k == _NTR_BASE

        def do_tile(k=k):
            t = pl.multiple_of((tr0 + k) * 8, 8)
            pltpu.sync_copy(idx_hbm.at[wid * 8 + k], idx8_v)
            for c in range(_NCHUNK):
                i = k * _NCHUNK + c
                b = i % _NBUF
                if i >= _NBUF:
                    drain(b)
                cin = pltpu.async_copy(
                    x_hbm.at[idx8_v, pl.ds(c * _W, _W)], bufs[b], isems[b])
                cin.wait()
                pltpu.async_copy(
                    bufs[b], out_hbm.at[pl.ds(t, 8), pl.ds(c * _W, _W)],
                    osems[b])

        if is_last:
            @pl.when(ntr > _NTR_BASE)
            def _():
                do_tile()
        else:
            do_tile()

    for b in range(_NBUF):
        drain(b)


def kernel(x1, x2, cell_ids):
    x2t = jnp.transpose(x2)
    out2t = _row_permute(x2t, jnp.asarray(_IDX_TAB))
    return (x1, jnp.transpose(out2t), cell_ids)


# speedup vs baseline: 1.1413x; 1.1413x over previous
import functools

import jax
import jax.numpy as jnp
import numpy as np
from jax import lax
from jax.experimental import pallas as pl
from jax.experimental.pallas import tpu as pltpu
from jax.experimental.pallas import tpu_sc as plsc

_N_FEAT = 2000
_BATCH = 8192

_NC = 2
_NS = 16
_NW = _NC * _NS
_N_TR = _N_FEAT // 8
_MAXK = 8

_SWAP_DST_SRC_2 = [
    (15, 462), (25, 1930), (26, 147), (43, 119), (59, 1744), (61, 964), (90, 167), (95, 1721),
    (114, 1123), (119, 43), (122, 1138), (126, 1965), (131, 426), (141, 908), (145, 1215), (147, 26),
    (156, 1437), (157, 1721), (167, 90), (173, 1276), (183, 997), (193, 1044), (227, 1925), (237, 1982),
    (242, 1400), (254, 1994), (273, 1139), (274, 1311), (277, 326), (286, 443), (318, 1618), (319, 1821),
    (326, 277), (338, 1058), (343, 996), (349, 1909), (357, 1013), (372, 1653), (380, 771), (393, 1834),
    (396, 683), (417, 669), (426, 131), (443, 286), (462, 15), (477, 616), (506, 969), (514, 1565),
    (515, 1103), (520, 1611), (550, 1224), (562, 1971), (566, 1333), (578, 1751), (582, 590), (584, 732),
    (589, 1894), (590, 582), (616, 477), (625, 1833), (634, 1623), (636, 646), (646, 636), (656, 1584),
    (669, 417), (680, 1724), (682, 1890), (683, 396), (690, 1617), (693, 1668), (699, 119), (732, 584),
    (741, 1807), (753, 1822), (754, 1162), (771, 380), (842, 879), (858, 1671), (879, 842), (888, 1800),
    (900, 1707), (908, 141), (911, 1515), (917, 1634), (927, 1948), (944, 1386), (945, 1022), (957, 1093),
    (961, 1330), (964, 61), (965, 1291), (969, 506), (996, 343), (997, 183), (1013, 357), (1022, 945),
    (1044, 193), (1058, 338), (1089, 1510), (1093, 957), (1103, 515), (1123, 114), (1138, 122), (1139, 273),
    (1162, 754), (1167, 1728), (1178, 1430), (1207, 1945), (1215, 145), (1224, 550), (1240, 1819), (1265, 1700),
    (1276, 173), (1291, 965), (1299, 1560), (1307, 1869), (1311, 274), (1323, 1968), (1330, 961), (1333, 566),
    (1348, 1497), (1359, 1607), (1381, 1412), (1386, 944), (1389, 1637), (1391, 1895), (1400, 242), (1412, 1381),
    (1417, 732), (1430, 1178), (1437, 156), (1455, 1644), (1465, 1651), (1480, 1752), (1497, 1348), (1510, 1089),
    (1515, 911), (1526, 1659), (1539, 1633), (1552, 1711), (1560, 1299), (1565, 514), (1584, 656), (1592, 1793),
    (1607, 1359), (1611, 520), (1617, 690), (1618, 318), (1623, 634), (1633, 1539), (1634, 917), (1637, 1389),
    (1644, 1455), (1651, 1465), (1653, 372), (1659, 1526), (1668, 693), (1671, 858), (1700, 1265), (1707, 900),
    (1711, 1552), (1721, 95), (1724, 680), (1728, 1167), (1744, 59), (1746, 1724), (1751, 578), (1752, 1480),
    (1793, 1592), (1800, 888), (1807, 741), (1819, 1240), (1821, 319), (1822, 753), (1833, 625), (1834, 393),
    (1869, 1307), (1890, 682), (1894, 589), (1895, 1391), (1909, 349), (1925, 227), (1930, 25), (1936, 1307),
    (1945, 1207), (1947, 147), (1948, 927), (1954, 242), (1965, 126), (1968, 1323), (1971, 562), (1982, 237),
    (1994, 254),
]


_W = 2048
_NCHUNK = _BATCH // _W
_NBUF = 3


def _idx_table(pairs):
    src_of = dict(pairs)
    ntrs = [_N_TR // _NW + (1 if w < _N_TR % _NW else 0) for w in range(_NW)]
    tab = np.zeros((_NW * 8, 8), dtype=np.int32)
    tr = 0
    for w in range(_NW):
        for k in range(ntrs[w]):
            t = tr + k
            tab[w * 8 + k] = [src_of.get(t * 8 + i, t * 8 + i)
                              for i in range(8)]
        tr += ntrs[w]
    return tab


_IDX_TAB = _idx_table(_SWAP_DST_SRC_2)
_NTR_BASE = _N_TR // _NW
_NTR_EXTRA = _N_TR % _NW

_mesh = plsc.VectorSubcoreMesh(core_axis_name="c", subcore_axis_name="s")


@functools.partial(
    pl.kernel,
    mesh=_mesh,
    compiler_params=pltpu.CompilerParams(
        needs_layout_passes=False, use_tc_tiling_on_sc=True),
    out_type=jax.ShapeDtypeStruct((_N_FEAT, _BATCH), jnp.float32),
    scratch_types=[
        pltpu.VMEM((8, _W), jnp.float32),
        pltpu.VMEM((8, _W), jnp.float32),
        pltpu.VMEM((8, _W), jnp.float32),
        pltpu.VMEM((2, 8), jnp.int32),
        pltpu.SemaphoreType.DMA,
        pltpu.SemaphoreType.DMA,
        pltpu.SemaphoreType.DMA,
        pltpu.SemaphoreType.DMA,
        pltpu.SemaphoreType.DMA,
        pltpu.SemaphoreType.DMA,
    ],
)
def _row_permute(x_hbm, idx_hbm, out_hbm,
                 buf_0, buf_1, buf_2, idx_v,
                 isem_0, isem_1, isem_2, osem_0, osem_1, osem_2):
    cid = lax.axis_index("c")
    sid = lax.axis_index("s")
    wid = sid * _NC + cid
    ntr = _NTR_BASE + jnp.where(wid < _NTR_EXTRA, 1, 0)
    tr0 = _NTR_BASE * wid + jnp.minimum(wid, _NTR_EXTRA)

    bufs = (buf_0, buf_1, buf_2)
    isems = (isem_0, isem_1, isem_2)
    osems = (osem_0, osem_1, osem_2)

    def drain(b):
        pltpu.make_async_copy(x_hbm.at[pl.ds(0, 8), pl.ds(0, _W)], bufs[b],
                              osems[b]).wait()

    def emit(n_tr):
        n = n_tr * _NCHUNK
        gathers = [None] * n

        def issue_gather(i):
            k, c = divmod(i, _NCHUNK)
            b = i % _NBUF
            if c == 0:
                pltpu.sync_copy(idx_hbm.at[wid * 8 + k], idx_v.at[k % 2])
            gathers[i] = pltpu.async_copy(
                x_hbm.at[idx_v.at[k % 2], pl.ds(c * _W, _W)],
                bufs[b], isems[b])

        def issue_out(i):
            k, c = divmod(i, _NCHUNK)
            b = i % _NBUF
            t = pl.multiple_of((tr0 + k) * 8, 8)
            gathers[i].wait()
            pltpu.async_copy(
                bufs[b], out_hbm.at[pl.ds(t, 8), pl.ds(c * _W, _W)], osems[b])

        for i in range(n):
            if i >= _NBUF:
                drain(i % _NBUF)
            issue_gather(i)
            if i >= 2:
                issue_out(i - 2)
        issue_out(n - 2)
        issue_out(n - 1)
        for b in range(_NBUF):
            drain(b)

    @pl.when(ntr == _NTR_BASE)
    def _():
        emit(_NTR_BASE)

    @pl.when(ntr == _NTR_BASE + 1)
    def _():
        emit(_NTR_BASE + 1)


def kernel(x1, x2, cell_ids):
    x2t = jnp.transpose(x2)
    out2t = _row_permute(x2t, jnp.asarray(_IDX_TAB))
    return (x1, jnp.transpose(out2t), cell_ids)


# speedup vs baseline: 1.1662x; 1.0218x over previous
import functools

import jax
import jax.numpy as jnp
import numpy as np
from jax import lax
from jax.experimental import pallas as pl
from jax.experimental.pallas import tpu as pltpu
from jax.experimental.pallas import tpu_sc as plsc

_N_FEAT = 2000
_BATCH = 8192

_NC = 2
_NS = 16
_NW = _NC * _NS
_N_TR = _N_FEAT // 8
_MAXK = 8

_SWAP_DST_SRC_2 = [
    (15, 462), (25, 1930), (26, 147), (43, 119), (59, 1744), (61, 964), (90, 167), (95, 1721),
    (114, 1123), (119, 43), (122, 1138), (126, 1965), (131, 426), (141, 908), (145, 1215), (147, 26),
    (156, 1437), (157, 1721), (167, 90), (173, 1276), (183, 997), (193, 1044), (227, 1925), (237, 1982),
    (242, 1400), (254, 1994), (273, 1139), (274, 1311), (277, 326), (286, 443), (318, 1618), (319, 1821),
    (326, 277), (338, 1058), (343, 996), (349, 1909), (357, 1013), (372, 1653), (380, 771), (393, 1834),
    (396, 683), (417, 669), (426, 131), (443, 286), (462, 15), (477, 616), (506, 969), (514, 1565),
    (515, 1103), (520, 1611), (550, 1224), (562, 1971), (566, 1333), (578, 1751), (582, 590), (584, 732),
    (589, 1894), (590, 582), (616, 477), (625, 1833), (634, 1623), (636, 646), (646, 636), (656, 1584),
    (669, 417), (680, 1724), (682, 1890), (683, 396), (690, 1617), (693, 1668), (699, 119), (732, 584),
    (741, 1807), (753, 1822), (754, 1162), (771, 380), (842, 879), (858, 1671), (879, 842), (888, 1800),
    (900, 1707), (908, 141), (911, 1515), (917, 1634), (927, 1948), (944, 1386), (945, 1022), (957, 1093),
    (961, 1330), (964, 61), (965, 1291), (969, 506), (996, 343), (997, 183), (1013, 357), (1022, 945),
    (1044, 193), (1058, 338), (1089, 1510), (1093, 957), (1103, 515), (1123, 114), (1138, 122), (1139, 273),
    (1162, 754), (1167, 1728), (1178, 1430), (1207, 1945), (1215, 145), (1224, 550), (1240, 1819), (1265, 1700),
    (1276, 173), (1291, 965), (1299, 1560), (1307, 1869), (1311, 274), (1323, 1968), (1330, 961), (1333, 566),
    (1348, 1497), (1359, 1607), (1381, 1412), (1386, 944), (1389, 1637), (1391, 1895), (1400, 242), (1412, 1381),
    (1417, 732), (1430, 1178), (1437, 156), (1455, 1644), (1465, 1651), (1480, 1752), (1497, 1348), (1510, 1089),
    (1515, 911), (1526, 1659), (1539, 1633), (1552, 1711), (1560, 1299), (1565, 514), (1584, 656), (1592, 1793),
    (1607, 1359), (1611, 520), (1617, 690), (1618, 318), (1623, 634), (1633, 1539), (1634, 917), (1637, 1389),
    (1644, 1455), (1651, 1465), (1653, 372), (1659, 1526), (1668, 693), (1671, 858), (1700, 1265), (1707, 900),
    (1711, 1552), (1721, 95), (1724, 680), (1728, 1167), (1744, 59), (1746, 1724), (1751, 578), (1752, 1480),
    (1793, 1592), (1800, 888), (1807, 741), (1819, 1240), (1821, 319), (1822, 753), (1833, 625), (1834, 393),
    (1869, 1307), (1890, 682), (1894, 589), (1895, 1391), (1909, 349), (1925, 227), (1930, 25), (1936, 1307),
    (1945, 1207), (1947, 147), (1948, 927), (1954, 242), (1965, 126), (1968, 1323), (1971, 562), (1982, 237),
    (1994, 254),
]


_W = 2048
_NCHUNK = _BATCH // _W
_NBUF = 3


def _idx_table(pairs):
    src_of = dict(pairs)
    ntrs = [_N_TR // _NW + (1 if w < _N_TR % _NW else 0) for w in range(_NW)]
    tab = np.zeros((_NW * 8, 8), dtype=np.int32)
    tr = 0
    for w in range(_NW):
        for k in range(ntrs[w]):
            t = tr + k
            tab[w * 8 + k] = [src_of.get(t * 8 + i, t * 8 + i)
                              for i in range(8)]
        tr += ntrs[w]
    return tab


_IDX_TAB = _idx_table(_SWAP_DST_SRC_2)
_NTR_BASE = _N_TR // _NW
_NTR_EXTRA = _N_TR % _NW

_mesh = plsc.VectorSubcoreMesh(core_axis_name="c", subcore_axis_name="s")


@functools.partial(
    pl.kernel,
    mesh=_mesh,
    compiler_params=pltpu.CompilerParams(
        needs_layout_passes=False, use_tc_tiling_on_sc=True),
    out_type=jax.ShapeDtypeStruct((_N_FEAT, _BATCH), jnp.float32),
    scratch_types=[
        pltpu.VMEM((8, _W), jnp.float32),
        pltpu.VMEM((8, _W), jnp.float32),
        pltpu.VMEM((8, _W), jnp.float32),
        pltpu.VMEM((2, 8), jnp.int32),
        pltpu.SemaphoreType.DMA,
        pltpu.SemaphoreType.DMA,
        pltpu.SemaphoreType.DMA,
        pltpu.SemaphoreType.DMA,
        pltpu.SemaphoreType.DMA,
        pltpu.SemaphoreType.DMA,
    ],
)
def _row_permute(x_hbm, idx_hbm, out_hbm,
                 buf_0, buf_1, buf_2, idx_v,
                 isem_0, isem_1, isem_2, osem_0, osem_1, osem_2):
    cid = lax.axis_index("c")
    sid = lax.axis_index("s")
    wid = sid * _NC + cid
    ntr = _NTR_BASE + jnp.where(wid < _NTR_EXTRA, 1, 0)
    tr0 = _NTR_BASE * wid + jnp.minimum(wid, _NTR_EXTRA)

    bufs = (buf_0, buf_1, buf_2)
    isems = (isem_0, isem_1, isem_2)
    osems = (osem_0, osem_1, osem_2)

    def drain(b):
        pltpu.make_async_copy(x_hbm.at[pl.ds(0, 8), pl.ds(0, _W)], bufs[b],
                              osems[b]).wait()

    def emit(n_tr):
        n = n_tr * _NCHUNK
        gathers = [None] * n

        def issue_gather(i):
            k, c = divmod(i, _NCHUNK)
            b = i % _NBUF
            if c == 0:
                pltpu.sync_copy(idx_hbm.at[wid * 8 + k], idx_v.at[k % 2])
            gathers[i] = pltpu.async_copy(
                x_hbm.at[idx_v.at[k % 2], pl.ds(c * _W, _W)],
                bufs[b], isems[b])

        def issue_out(i):
            k, c = divmod(i, _NCHUNK)
            b = i % _NBUF
            t = pl.multiple_of((tr0 + k) * 8, 8)
            gathers[i].wait()
            pltpu.async_copy(
                bufs[b], out_hbm.at[pl.ds(t, 8), pl.ds(c * _W, _W)], osems[b])

        for i in range(n):
            if i >= _NBUF:
                drain(i % _NBUF)
            issue_gather(i)
            if i >= 2:
                issue_out(i - 2)
        issue_out(n - 2)
        issue_out(n - 1)
        for b in range(_NBUF):
            drain(b)

    @pl.when(ntr == _NTR_BASE)
    def _():
        emit(_NTR_BASE)

    @pl.when(ntr == _NTR_BASE + 1)
    def _():
        emit(_NTR_BASE + 1)


_COPY_ROWS = 40


def _tc_copy(xt):
    def body(x_ref, o_ref):
        o_ref[...] = x_ref[...]

    return pl.pallas_call(
        body,
        grid=(_N_FEAT // _COPY_ROWS,),
        in_specs=[pl.BlockSpec((_COPY_ROWS, _BATCH), lambda i: (i, 0))],
        out_specs=pl.BlockSpec((_COPY_ROWS, _BATCH), lambda i: (i, 0)),
        out_shape=jax.ShapeDtypeStruct((_N_FEAT, _BATCH), jnp.float32),
    )(xt)


def kernel(x1, x2, cell_ids):
    x2t = jnp.transpose(x2)
    out2t = _row_permute(x2t, jnp.asarray(_IDX_TAB))
    out1t = _tc_copy(jnp.transpose(x1))
    return (jnp.transpose(out1t), jnp.transpose(out2t), cell_ids)


# speedup vs baseline: 1.2021x; 1.0308x over previous
import functools

import jax
import jax.numpy as jnp
import numpy as np
from jax import lax
from jax.experimental import pallas as pl
from jax.experimental.pallas import tpu as pltpu
from jax.experimental.pallas import tpu_sc as plsc

_N_FEAT = 2000
_BATCH = 8192

_NC = 2
_NS = 16
_NW = _NC * _NS
_N_TR = _N_FEAT // 8

_SWAP_DST_SRC_2 = [
    (15, 462), (25, 1930), (26, 147), (43, 119), (59, 1744), (61, 964), (90, 167), (95, 1721),
    (114, 1123), (119, 43), (122, 1138), (126, 1965), (131, 426), (141, 908), (145, 1215), (147, 26),
    (156, 1437), (157, 1721), (167, 90), (173, 1276), (183, 997), (193, 1044), (227, 1925), (237, 1982),
    (242, 1400), (254, 1994), (273, 1139), (274, 1311), (277, 326), (286, 443), (318, 1618), (319, 1821),
    (326, 277), (338, 1058), (343, 996), (349, 1909), (357, 1013), (372, 1653), (380, 771), (393, 1834),
    (396, 683), (417, 669), (426, 131), (443, 286), (462, 15), (477, 616), (506, 969), (514, 1565),
    (515, 1103), (520, 1611), (550, 1224), (562, 1971), (566, 1333), (578, 1751), (582, 590), (584, 732),
    (589, 1894), (590, 582), (616, 477), (625, 1833), (634, 1623), (636, 646), (646, 636), (656, 1584),
    (669, 417), (680, 1724), (682, 1890), (683, 396), (690, 1617), (693, 1668), (699, 119), (732, 584),
    (741, 1807), (753, 1822), (754, 1162), (771, 380), (842, 879), (858, 1671), (879, 842), (888, 1800),
    (900, 1707), (908, 141), (911, 1515), (917, 1634), (927, 1948), (944, 1386), (945, 1022), (957, 1093),
    (961, 1330), (964, 61), (965, 1291), (969, 506), (996, 343), (997, 183), (1013, 357), (1022, 945),
    (1044, 193), (1058, 338), (1089, 1510), (1093, 957), (1103, 515), (1123, 114), (1138, 122), (1139, 273),
    (1162, 754), (1167, 1728), (1178, 1430), (1207, 1945), (1215, 145), (1224, 550), (1240, 1819), (1265, 1700),
    (1276, 173), (1291, 965), (1299, 1560), (1307, 1869), (1311, 274), (1323, 1968), (1330, 961), (1333, 566),
    (1348, 1497), (1359, 1607), (1381, 1412), (1386, 944), (1389, 1637), (1391, 1895), (1400, 242), (1412, 1381),
    (1417, 732), (1430, 1178), (1437, 156), (1455, 1644), (1465, 1651), (1480, 1752), (1497, 1348), (1510, 1089),
    (1515, 911), (1526, 1659), (1539, 1633), (1552, 1711), (1560, 1299), (1565, 514), (1584, 656), (1592, 1793),
    (1607, 1359), (1611, 520), (1617, 690), (1618, 318), (1623, 634), (1633, 1539), (1634, 917), (1637, 1389),
    (1644, 1455), (1651, 1465), (1653, 372), (1659, 1526), (1668, 693), (1671, 858), (1700, 1265), (1707, 900),
    (1711, 1552), (1721, 95), (1724, 680), (1728, 1167), (1744, 59), (1746, 1724), (1751, 578), (1752, 1480),
    (1793, 1592), (1800, 888), (1807, 741), (1819, 1240), (1821, 319), (1822, 753), (1833, 625), (1834, 393),
    (1869, 1307), (1890, 682), (1894, 589), (1895, 1391), (1909, 349), (1925, 227), (1930, 25), (1936, 1307),
    (1945, 1207), (1947, 147), (1948, 927), (1954, 242), (1965, 126), (1968, 1323), (1971, 562), (1982, 237),
    (1994, 254),
]


_W = 2048
_NCHUNK = _BATCH // _W
_NBUF = 3


def _idx_table(pairs):
    src_of = dict(pairs)
    ntrs = [_N_TR // _NW + (1 if w < _N_TR % _NW else 0) for w in range(_NW)]
    tab = np.zeros((_NW * 8, 8), dtype=np.int32)
    tr = 0
    for w in range(_NW):
        for k in range(ntrs[w]):
            t = tr + k
            tab[w * 8 + k] = [src_of.get(t * 8 + i, t * 8 + i)
                              for i in range(8)]
        tr += ntrs[w]
    return tab


_IDX_TAB = _idx_table(_SWAP_DST_SRC_2)
_NTR_BASE = _N_TR // _NW
_NTR_EXTRA = _N_TR % _NW

_mesh = plsc.VectorSubcoreMesh(core_axis_name="c", subcore_axis_name="s")


@functools.partial(
    pl.kernel,
    mesh=_mesh,
    compiler_params=pltpu.CompilerParams(
        needs_layout_passes=False, use_tc_tiling_on_sc=True),
    out_type=jax.ShapeDtypeStruct((_N_FEAT, _BATCH), jnp.float32),
    scratch_types=[
        pltpu.VMEM((8, _W), jnp.float32),
        pltpu.VMEM((8, _W), jnp.float32),
        pltpu.VMEM((8, _W), jnp.float32),
        pltpu.VMEM((2, 8), jnp.int32),
        pltpu.SemaphoreType.DMA,
        pltpu.SemaphoreType.DMA,
        pltpu.SemaphoreType.DMA,
        pltpu.SemaphoreType.DMA,
        pltpu.SemaphoreType.DMA,
        pltpu.SemaphoreType.DMA,
    ],
)
def _row_permute(x_hbm, idx_hbm, out_hbm,
                 buf_0, buf_1, buf_2, idx_v,
                 isem_0, isem_1, isem_2, osem_0, osem_1, osem_2):
    cid = lax.axis_index("c")
    sid = lax.axis_index("s")
    wid = sid * _NC + cid
    ntr = _NTR_BASE + jnp.where(wid < _NTR_EXTRA, 1, 0)
    tr0 = _NTR_BASE * wid + jnp.minimum(wid, _NTR_EXTRA)

    bufs = (buf_0, buf_1, buf_2)
    isems = (isem_0, isem_1, isem_2)
    osems = (osem_0, osem_1, osem_2)

    def drain(b):
        pltpu.make_async_copy(x_hbm.at[pl.ds(0, 8), pl.ds(0, _W)], bufs[b],
                              osems[b]).wait()

    def emit(n_tr):
        n = n_tr * _NCHUNK
        gathers = [None] * n

        def issue_gather(i):
            k, c = divmod(i, _NCHUNK)
            b = i % _NBUF
            if c == 0:
                pltpu.sync_copy(idx_hbm.at[wid * 8 + k], idx_v.at[k % 2])
            gathers[i] = pltpu.async_copy(
                x_hbm.at[idx_v.at[k % 2], pl.ds(c * _W, _W)],
                bufs[b], isems[b])

        def issue_out(i):
            k, c = divmod(i, _NCHUNK)
            b = i % _NBUF
            t = pl.multiple_of((tr0 + k) * 8, 8)
            gathers[i].wait()
            pltpu.async_copy(
                bufs[b], out_hbm.at[pl.ds(t, 8), pl.ds(c * _W, _W)], osems[b])

        for i in range(n):
            if i >= _NBUF:
                drain(i % _NBUF)
            issue_gather(i)
            if i >= 2:
                issue_out(i - 2)
        issue_out(n - 2)
        issue_out(n - 1)
        for b in range(_NBUF):
            drain(b)

    @pl.when(ntr == _NTR_BASE)
    def _():
        emit(_NTR_BASE)

    @pl.when(ntr == _NTR_BASE + 1)
    def _():
        emit(_NTR_BASE + 1)


_COPY_ROWS = 200


def _tc_copy(xt):
    def body(x_ref, o_ref):
        o_ref[...] = x_ref[...]

    return pl.pallas_call(
        body,
        grid=(_N_FEAT // _COPY_ROWS,),
        in_specs=[pl.BlockSpec((_COPY_ROWS, _BATCH), lambda i: (i, 0))],
        out_specs=pl.BlockSpec((_COPY_ROWS, _BATCH), lambda i: (i, 0)),
        out_shape=jax.ShapeDtypeStruct((_N_FEAT, _BATCH), jnp.float32),
    )(xt)


def kernel(x1, x2, cell_ids):
    x2t = jnp.transpose(x2)
    out2t = _row_permute(x2t, jnp.asarray(_IDX_TAB))
    out1t = _tc_copy(jnp.transpose(x1))
    return (jnp.transpose(out1t), jnp.transpose(out2t), cell_ids)
